# baseline (speedup 1.0000x reference)
import jax
import jax.numpy as jnp
from jax.experimental import pallas as pl
from jax.experimental.pallas import tpu as pltpu

_TILE = 8192


def _fused_kernel(x_ref, w_ref, b_ref, out_ref):
    z = jnp.dot(x_ref[...], w_ref[...], preferred_element_type=jnp.float32)
    out_ref[...] = jnp.maximum(z + b_ref[...], 0.0)


def kernel(x, beta, w, b):
    n, d = x.shape
    e = w.shape[1]
    w_eff = beta * w
    b_row = b.reshape(1, e)

    p = 128 // d if (d < 128 and 128 % d == 0) else 1
    if p > 1:
        b_p = jnp.tile(b_row, (1, p))
        w_p = jax.scipy.linalg.block_diag(*([w_eff] * p))
        dp, ep = p * d, p * e
    else:
        b_p, w_p, dp, ep = b_row, w_eff, d, e

    rows = n // p
    tile = min(_TILE, ((rows + 7) // 8) * 8)
    rows_pad = ((rows + tile - 1) // tile) * tile
    xp = x.reshape(rows, dp)
    if rows_pad != rows:
        xp = jnp.pad(xp, ((0, rows_pad - rows), (0, 0)))

    out = pl.pallas_call(
        _fused_kernel,
        out_shape=jax.ShapeDtypeStruct((rows_pad, ep), jnp.float32),
        grid=(rows_pad // tile,),
        in_specs=[
            pl.BlockSpec((tile, dp), lambda i: (i, 0)),
            pl.BlockSpec((dp, ep), lambda i: (0, 0)),
            pl.BlockSpec((1, ep), lambda i: (0, 0)),
        ],
        out_specs=pl.BlockSpec((tile, ep), lambda i: (i, 0)),
        compiler_params=pltpu.CompilerParams(
            dimension_semantics=("parallel",),
            skip_device_barrier=True),
    )(xp, w_p, b_p)
    return out[:rows].reshape(n, e)


# speedup vs baseline: 1.2093x; 1.2093x over previous
import jax
import jax.numpy as jnp
from jax.experimental import pallas as pl
from jax.experimental.pallas import tpu as pltpu

_CHUNK = 8192
_SLOTS = 4


def _make_mdma_kernel(n_chunks, chunk):
    def body(x_hbm, w_ref, b_ref, o_hbm, x_buf, o_buf, in_sems, out_sems):
        def start_in(slot, t):
            pltpu.make_async_copy(
                x_hbm.at[pl.ds(t * chunk, chunk)], x_buf.at[slot],
                in_sems.at[slot]).start()

        def wait_in(slot):
            pltpu.make_async_copy(
                x_hbm.at[pl.ds(0, chunk)], x_buf.at[slot],
                in_sems.at[slot]).wait()

        def start_out(slot, t):
            pltpu.make_async_copy(
                o_buf.at[slot], o_hbm.at[pl.ds(t * chunk, chunk)],
                out_sems.at[slot]).start()

        def wait_out(slot):
            pltpu.make_async_copy(
                o_buf.at[slot], o_hbm.at[pl.ds(0, chunk)],
                out_sems.at[slot]).wait()

        for s in range(min(_SLOTS, n_chunks)):
            start_in(s, s)

        def step(t, carry):
            slot = jax.lax.rem(t, _SLOTS)
            wait_in(slot)

            @pl.when(t >= _SLOTS)
            def _():
                wait_out(slot)

            z = jnp.dot(x_buf[slot], w_ref[...],
                        preferred_element_type=jnp.float32)
            o_buf[slot] = jnp.maximum(z + b_ref[...], 0.0)
            start_out(slot, t)

            @pl.when(t + _SLOTS < n_chunks)
            def _():
                start_in(slot, t + _SLOTS)

            return carry

        jax.lax.fori_loop(0, n_chunks, step, 0)

        for s in range(min(_SLOTS, n_chunks)):
            wait_out(s)

    return body


def kernel(x, beta, w, b):
    n, d = x.shape
    e = w.shape[1]
    w_eff = beta * w
    b_row = b.reshape(1, e)

    chunk = _CHUNK if n % _CHUNK == 0 and n // _CHUNK >= _SLOTS else None
    if chunk is None:
        tile = min(8192, ((n + 7) // 8) * 8)
        n_pad = ((n + tile - 1) // tile) * tile
        xp = jnp.pad(x, ((0, n_pad - n), (0, 0))) if n_pad != n else x
        out = pl.pallas_call(
            lambda x_ref, w_ref, b_ref, o_ref: o_ref.__setitem__(
                ..., jnp.maximum(
                    jnp.dot(x_ref[...], w_ref[...],
                            preferred_element_type=jnp.float32) + b_ref[...],
                    0.0)),
            out_shape=jax.ShapeDtypeStruct((n_pad, e), jnp.float32),
            grid=(n_pad // tile,),
            in_specs=[
                pl.BlockSpec((tile, d), lambda i: (i, 0)),
                pl.BlockSpec((d, e), lambda i: (0, 0)),
                pl.BlockSpec((1, e), lambda i: (0, 0)),
            ],
            out_specs=pl.BlockSpec((tile, e), lambda i: (i, 0)),
            compiler_params=pltpu.CompilerParams(
                dimension_semantics=("parallel",)),
        )(xp, w_eff, b_row)
        return out[:n]

    n_chunks = n // chunk
    return pl.pallas_call(
        _make_mdma_kernel(n_chunks, chunk),
        out_shape=jax.ShapeDtypeStruct((n, e), jnp.float32),
        in_specs=[
            pl.BlockSpec(memory_space=pl.ANY),
            pl.BlockSpec(memory_space=pltpu.VMEM),
            pl.BlockSpec(memory_space=pltpu.VMEM),
        ],
        out_specs=pl.BlockSpec(memory_space=pl.ANY),
        scratch_shapes=[
            pltpu.VMEM((_SLOTS, chunk, d), jnp.float32),
            pltpu.VMEM((_SLOTS, chunk, e), jnp.float32),
            pltpu.SemaphoreType.DMA((_SLOTS,)),
            pltpu.SemaphoreType.DMA((_SLOTS,)),
        ],
    )(x, w_eff, b_row)
